# baseline (speedup 1.0000x reference)
import functools

import jax
import jax.numpy as jnp
from jax import lax
from jax.experimental import pallas as pl
from jax.experimental.pallas import tpu as pltpu
from jax.experimental.pallas import tpu_sc as plsc

N = 8192
D = 64
E_SIG = 32768
E_RND = 65536
E_ALL = E_SIG + E_RND
K1 = 17
MARGIN = 1.0
R_MAX = 100.0

BR = 256
NSTRIPES = N // BR
NC2 = 2048
BE = 4096
NEBLK = E_ALL // BE


def _knn_body(emb_row_ref, emb_all_ref, pid_row_ref, pid_col_ref, out_ref,
              stripe_ref, cand_ref, sqc_ref):
    i = pl.program_id(0)
    a = emb_row_ref[...]
    b = emb_all_ref[...]
    dots = jax.lax.dot_general(
        a, b, (((1,), (1,)), ((), ())),
        preferred_element_type=jnp.float32,
        precision=jax.lax.Precision.HIGHEST)
    sq_c = jax.lax.dot_general(
        jnp.ones((8, D), jnp.float32), b * b, (((1,), (1,)), ((), ())),
        preferred_element_type=jnp.float32,
        precision=jax.lax.Precision.HIGHEST)[:1]
    stripe_ref[...] = dots
    sqc_ref[...] = sq_c

    sub8 = jax.lax.broadcasted_iota(jnp.int32, (8, 128), 0)
    lane8 = jax.lax.broadcasted_iota(jnp.int32, (8, 128), 1)

    def rowblk(rb, _):
        a8 = emb_row_ref[pl.ds(rb * 8, 8), :]
        sqr8 = jnp.sum(a8 * a8, axis=1, keepdims=True)
        pidr8 = pid_row_ref[pl.ds(rb * 8, 8), :]
        row8 = i * BR + rb * 8 + sub8

        def load(t):
            co = t * 128
            dl = stripe_ref[pl.ds(rb * 8, 8), pl.ds(co, 128)]
            sqc8 = sqc_ref[:, pl.ds(co, 128)]
            pidc8 = pid_col_ref[:, pl.ds(co, 128)]
            pmi = jnp.where((pidr8 == pidc8) & (pidr8 != 0), 1, 0)
            pk = jax.lax.bitcast_convert_type(
                (jax.lax.bitcast_convert_type(
                    jnp.maximum(sqr8 + sqc8 - 2.0 * dl, 0.0), jnp.int32)
                 & -2) | pmi, jnp.float32)
            return jnp.where(co + lane8 == row8, jnp.float32(3.0e38), pk)

        Rs = None
        for g in range(4):
            G = [load(g * 16 + t) for t in range(16)]
            for k in (2, 4, 8, 16):
                j = k >> 1
                while j >= 1:
                    for idx in range(16):
                        l = idx ^ j
                        if l > idx:
                            a, b = G[idx], G[l]
                            lo = jnp.minimum(a, b)
                            hi = jnp.maximum(a, b)
                            if (idx & k) == 0:
                                G[idx], G[l] = lo, hi
                            else:
                                G[idx], G[l] = hi, lo
                    j >>= 1
            if Rs is None:
                Rs = G
            else:
                M = [jnp.minimum(Rs[t], G[15 - t]) for t in range(16)]
                if g < 3:
                    for j in (8, 4, 2, 1):
                        for idx in range(16):
                            l = idx ^ j
                            if l > idx:
                                a, b = M[idx], M[l]
                                M[idx], M[l] = (jnp.minimum(a, b),
                                                jnp.maximum(a, b))
                Rs = M
        for t in range(16):
            cand_ref[pl.ds(rb * 8, 8), pl.ds(t * 128, 128)] = Rs[t]
        return 0

    jax.lax.fori_loop(0, BR // 8, rowblk, 0)

    colio2 = jax.lax.broadcasted_iota(jnp.int32, (BR, NC2), 1)

    def body(_, carry):
        num, den = carry
        s = cand_ref[...]
        vmin = jnp.min(s, axis=1, keepdims=True)
        amin = jnp.min(jnp.where(s == vmin, colio2, jnp.int32(NC2)),
                       axis=1, keepdims=True)
        cand_ref[...] = jnp.where(colio2 == amin, jnp.float32(3.0e38), s)
        pmv = jax.lax.bitcast_convert_type(vmin, jnp.int32) & 1
        d = jnp.sqrt(vmin + 1e-12)
        mf = jnp.where(d <= R_MAX, 1.0, 0.0)
        l = jnp.where(pmv == 1, d, jnp.maximum(0.0, MARGIN - d))
        return num + l * mf, den + mf

    zero = jnp.zeros((BR, 1), jnp.float32)
    num, den = jax.lax.fori_loop(0, K1 - 1, body, (zero, zero))
    lane = jax.lax.broadcasted_iota(jnp.int32, (1, 1, 128), 2)
    out_ref[...] = jnp.where(lane == 0, jnp.sum(num),
                             jnp.where(lane == 1, jnp.sum(den), 0.0))


def _edge_body(es_ref, ed_ref, wsig_ref, hs_ref, pid_ref, out_ref):
    i = pl.program_id(0)
    es = es_ref[...]
    ed = ed_ref[...]
    laneio = jax.lax.broadcasted_iota(jnp.int32, (BE, DP), 1)
    diff = jnp.where(laneio < D, es - ed, 0.0)
    d = jnp.sqrt(jnp.sum(diff * diff, axis=1, keepdims=True) + 1e-12)
    ps = jax.lax.bitcast_convert_type(es[:, D:D + 1], jnp.int32)
    pd = jax.lax.bitcast_convert_type(ed[:, D:D + 1], jnp.int32)
    y = (ps == pd) & (ps != 0)
    l = jnp.where(y, d, jnp.maximum(0.0, MARGIN - d))
    w = wsig_ref[...]
    ssum = jnp.sum(l * w)
    rsum = jnp.sum(l * (1.0 - w))

    def beta():
        x = hs_ref[...]
        t = jnp.where(pid_ref[...] != 0, 1.0, 0.0)
        bce = (jnp.maximum(x, 0.0) - x * t
               + jnp.log1p(jnp.exp(-jnp.abs(x))))
        return jnp.sum(bce)

    bsum = jax.lax.cond(i == 0, beta, lambda: jnp.float32(0.0))
    lane = jax.lax.broadcasted_iota(jnp.int32, (1, 1, 128), 2)
    out_ref[...] = jnp.where(lane == 0, ssum,
                             jnp.where(lane == 1, rsum,
                                       jnp.where(lane == 2, bsum, 0.0)))


NW = 32
EPW = E_ALL // NW
ECH = 384
NCH = EPW // ECH
DP = 128


def _sc_gather_body(tab_hbm, src_hbm, dst_hbm, es_hbm, ed_hbm,
                    idxs_v, idxd_v, rows_s, rows_d, sem):
    wid = lax.axis_index("s") * 2 + lax.axis_index("c")

    def chunk(c, _):
        base = wid * EPW + c * ECH
        pltpu.sync_copy(src_hbm.at[pl.ds(base, ECH)], idxs_v)
        pltpu.sync_copy(dst_hbm.at[pl.ds(base, ECH)], idxd_v)
        cp1 = pltpu.async_copy(tab_hbm.at[idxs_v], rows_s, sem)
        cp2 = pltpu.async_copy(tab_hbm.at[idxd_v], rows_d, sem)
        cp1.wait()
        cp2.wait()
        pltpu.sync_copy(rows_s, es_hbm.at[pl.ds(base, ECH)])
        pltpu.sync_copy(rows_d, ed_hbm.at[pl.ds(base, ECH)])
        return 0

    lax.fori_loop(0, NCH, chunk, 0)


def _sc_gather(tab, src, dst):
    mesh = plsc.VectorSubcoreMesh(core_axis_name="c", subcore_axis_name="s")
    f = functools.partial(
        pl.kernel, mesh=mesh,
        out_type=[
            jax.ShapeDtypeStruct((E_ALL, DP), jnp.float32),
            jax.ShapeDtypeStruct((E_ALL, DP), jnp.float32),
        ],
        scratch_types=[
            pltpu.VMEM((ECH,), jnp.int32),
            pltpu.VMEM((ECH,), jnp.int32),
            pltpu.VMEM((ECH, DP), jnp.float32),
            pltpu.VMEM((ECH, DP), jnp.float32),
            pltpu.SemaphoreType.DMA,
        ],
    )(_sc_gather_body)
    return f(tab, src, dst)


@functools.partial(jax.jit, static_argnames=())
def kernel(embeddings, hit_score, hit_particle_id, signal_edges, random_edges):
    emb = embeddings.astype(jnp.float32)
    pid = hit_particle_id.astype(jnp.int32)
    pid_row = pid.reshape(N, 1)
    pid_col = pid.reshape(1, N)

    knn_part = pl.pallas_call(
        _knn_body,
        grid=(NSTRIPES,),
        in_specs=[
            pl.BlockSpec((BR, D), lambda i: (i, 0)),
            pl.BlockSpec((N, D), lambda i: (0, 0)),
            pl.BlockSpec((BR, 1), lambda i: (i, 0)),
            pl.BlockSpec((1, N), lambda i: (0, 0)),
        ],
        out_specs=pl.BlockSpec((1, 1, 128), lambda i: (i, 0, 0)),
        out_shape=jax.ShapeDtypeStruct((NSTRIPES, 1, 128), jnp.float32),
        scratch_shapes=[
            pltpu.VMEM((BR, N), jnp.float32),
            pltpu.VMEM((BR, NC2), jnp.float32),
            pltpu.VMEM((1, N), jnp.float32),
        ],
    )(emb, emb, pid_row, pid_col)

    src = jnp.concatenate([signal_edges[0], random_edges[0]]).astype(jnp.int32)
    dst = jnp.concatenate([signal_edges[1], random_edges[1]]).astype(jnp.int32)
    tab = jnp.concatenate(
        [emb, jax.lax.bitcast_convert_type(pid, jnp.float32).reshape(N, 1),
         jnp.zeros((N, DP - D - 1), jnp.float32)], axis=1)
    es, ed = _sc_gather(tab, src, dst)
    wsig = (jnp.arange(E_ALL) < E_SIG).astype(jnp.float32).reshape(E_ALL, 1)

    edge_part = pl.pallas_call(
        _edge_body,
        grid=(NEBLK,),
        in_specs=[
            pl.BlockSpec((BE, DP), lambda i: (i, 0)),
            pl.BlockSpec((BE, DP), lambda i: (i, 0)),
            pl.BlockSpec((BE, 1), lambda i: (i, 0)),
            pl.BlockSpec((N, 1), lambda i: (0, 0)),
            pl.BlockSpec((N, 1), lambda i: (0, 0)),
        ],
        out_specs=pl.BlockSpec((1, 1, 128), lambda i: (i, 0, 0)),
        out_shape=jax.ShapeDtypeStruct((NEBLK, 1, 128), jnp.float32),
    )(es, ed, wsig, hit_score.astype(jnp.float32).reshape(N, 1), pid_row)

    knn_num = jnp.sum(knn_part[:, 0, 0])
    knn_den = jnp.sum(knn_part[:, 0, 1])
    knn_loss = knn_num / jnp.maximum(knn_den, 1.0)
    sig_sum = jnp.sum(edge_part[:, 0, 0])
    rnd_sum = jnp.sum(edge_part[:, 0, 1])
    beta_sum = jnp.sum(edge_part[:, 0, 2])
    signal_loss = sig_sum / float(E_SIG)
    random_loss = rnd_sum / float(E_RND)
    beta_loss = beta_sum / float(N)
    total = signal_loss + knn_loss + random_loss + beta_loss
    return jnp.stack([total, signal_loss, knn_loss, random_loss, beta_loss])


# speedup vs baseline: 1.0659x; 1.0659x over previous
import functools

import jax
import jax.numpy as jnp
from jax import lax
from jax.experimental import pallas as pl
from jax.experimental.pallas import tpu as pltpu
from jax.experimental.pallas import tpu_sc as plsc

N = 8192
D = 64
E_SIG = 32768
E_RND = 65536
E_ALL = E_SIG + E_RND
K1 = 17
MARGIN = 1.0
R_MAX = 100.0

BR = 256
NSTRIPES = N // BR
NC2 = 2048
BE = 4096
NEBLK = E_ALL // BE


def _knn_body(emb_row_ref, emb_all_ref, pid_row_ref, pid_col_ref, out_ref,
              stripe_ref, cand_ref):
    i = pl.program_id(0)
    a = emb_row_ref[...]
    b = emb_all_ref[...]
    dots = jax.lax.dot_general(
        a, b, (((1,), (1,)), ((), ())),
        preferred_element_type=jnp.float32,
        precision=jax.lax.Precision.HIGHEST)
    sq_r = jnp.sum(a * a, axis=1, keepdims=True)
    sq_c = jax.lax.dot_general(
        jnp.ones((8, D), jnp.float32), b * b, (((1,), (1,)), ((), ())),
        preferred_element_type=jnp.float32,
        precision=jax.lax.Precision.HIGHEST)[:1]
    colio = jax.lax.broadcasted_iota(jnp.int32, (BR, N), 1)
    rowid = i * BR + jax.lax.broadcasted_iota(jnp.int32, (BR, 1), 0)
    pid_r = pid_row_ref[...]
    pid_c = pid_col_ref[...]
    pmi = jnp.where((pid_r == pid_c) & (pid_r != 0), 1, 0)
    packed = jax.lax.bitcast_convert_type(
        (jax.lax.bitcast_convert_type(
            jnp.maximum(sq_r + sq_c - 2.0 * dots, 0.0), jnp.int32) & -2)
        | pmi, jnp.float32)
    stripe_ref[...] = jnp.where(colio == rowid, jnp.float32(3.0e38), packed)

    def rowblk(rb, _):
        Rs = None
        for g in range(4):
            G = [stripe_ref[pl.ds(rb * 8, 8), pl.ds((g * 16 + t) * 128, 128)]
                 for t in range(16)]
            for k in (2, 4, 8, 16):
                j = k >> 1
                while j >= 1:
                    for idx in range(16):
                        l = idx ^ j
                        if l > idx:
                            a, b = G[idx], G[l]
                            lo = jnp.minimum(a, b)
                            hi = jnp.maximum(a, b)
                            if (idx & k) == 0:
                                G[idx], G[l] = lo, hi
                            else:
                                G[idx], G[l] = hi, lo
                    j >>= 1
            if Rs is None:
                Rs = G
            else:
                M = [jnp.minimum(Rs[t], G[15 - t]) for t in range(16)]
                if g < 3:
                    for j in (8, 4, 2, 1):
                        for idx in range(16):
                            l = idx ^ j
                            if l > idx:
                                a, b = M[idx], M[l]
                                M[idx], M[l] = (jnp.minimum(a, b),
                                                jnp.maximum(a, b))
                Rs = M
        for t in range(16):
            cand_ref[pl.ds(rb * 8, 8), pl.ds(t * 128, 128)] = Rs[t]
        return 0

    jax.lax.fori_loop(0, BR // 8, rowblk, 0)

    colio2 = jax.lax.broadcasted_iota(jnp.int32, (BR, NC2), 1)

    def body(_, carry):
        num, den = carry
        s = cand_ref[...]
        vmin = jnp.min(s, axis=1, keepdims=True)
        amin = jnp.min(jnp.where(s == vmin, colio2, jnp.int32(NC2)),
                       axis=1, keepdims=True)
        cand_ref[...] = jnp.where(colio2 == amin, jnp.float32(3.0e38), s)
        pmv = jax.lax.bitcast_convert_type(vmin, jnp.int32) & 1
        d = jnp.sqrt(vmin + 1e-12)
        mf = jnp.where(d <= R_MAX, 1.0, 0.0)
        l = jnp.where(pmv == 1, d, jnp.maximum(0.0, MARGIN - d))
        return num + l * mf, den + mf

    zero = jnp.zeros((BR, 1), jnp.float32)
    num, den = jax.lax.fori_loop(0, K1 - 1, body, (zero, zero))
    lane = jax.lax.broadcasted_iota(jnp.int32, (1, 1, 128), 2)
    out_ref[...] = jnp.where(lane == 0, jnp.sum(num),
                             jnp.where(lane == 1, jnp.sum(den), 0.0))


def _edge_body(es_ref, ed_ref, wsig_ref, hs_ref, pid_ref, out_ref):
    i = pl.program_id(0)
    es = es_ref[...]
    ed = ed_ref[...]
    laneio = jax.lax.broadcasted_iota(jnp.int32, (BE, DP), 1)
    diff = jnp.where(laneio < D, es - ed, 0.0)
    d = jnp.sqrt(jnp.sum(diff * diff, axis=1, keepdims=True) + 1e-12)
    ps = jax.lax.bitcast_convert_type(es[:, D:D + 1], jnp.int32)
    pd = jax.lax.bitcast_convert_type(ed[:, D:D + 1], jnp.int32)
    y = (ps == pd) & (ps != 0)
    l = jnp.where(y, d, jnp.maximum(0.0, MARGIN - d))
    w = wsig_ref[...]
    ssum = jnp.sum(l * w)
    rsum = jnp.sum(l * (1.0 - w))

    def beta():
        x = hs_ref[...]
        t = jnp.where(pid_ref[...] != 0, 1.0, 0.0)
        bce = (jnp.maximum(x, 0.0) - x * t
               + jnp.log1p(jnp.exp(-jnp.abs(x))))
        return jnp.sum(bce)

    bsum = jax.lax.cond(i == 0, beta, lambda: jnp.float32(0.0))
    lane = jax.lax.broadcasted_iota(jnp.int32, (1, 1, 128), 2)
    out_ref[...] = jnp.where(lane == 0, ssum,
                             jnp.where(lane == 1, rsum,
                                       jnp.where(lane == 2, bsum, 0.0)))


NW = 32
EPW = E_ALL // NW
ECH = 384
NCH = EPW // ECH
DP = 128


def _sc_gather_body(tab_hbm, src_hbm, dst_hbm, es_hbm, ed_hbm,
                    idxs_v, idxd_v, rows_s, rows_d, sem):
    wid = lax.axis_index("s") * 2 + lax.axis_index("c")

    def chunk(c, _):
        base = wid * EPW + c * ECH
        pltpu.sync_copy(src_hbm.at[pl.ds(base, ECH)], idxs_v)
        pltpu.sync_copy(dst_hbm.at[pl.ds(base, ECH)], idxd_v)
        cp1 = pltpu.async_copy(tab_hbm.at[idxs_v], rows_s, sem)
        cp2 = pltpu.async_copy(tab_hbm.at[idxd_v], rows_d, sem)
        cp1.wait()
        cp2.wait()
        pltpu.sync_copy(rows_s, es_hbm.at[pl.ds(base, ECH)])
        pltpu.sync_copy(rows_d, ed_hbm.at[pl.ds(base, ECH)])
        return 0

    lax.fori_loop(0, NCH, chunk, 0)


def _sc_gather(tab, src, dst):
    mesh = plsc.VectorSubcoreMesh(core_axis_name="c", subcore_axis_name="s")
    f = functools.partial(
        pl.kernel, mesh=mesh,
        out_type=[
            jax.ShapeDtypeStruct((E_ALL, DP), jnp.float32),
            jax.ShapeDtypeStruct((E_ALL, DP), jnp.float32),
        ],
        scratch_types=[
            pltpu.VMEM((ECH,), jnp.int32),
            pltpu.VMEM((ECH,), jnp.int32),
            pltpu.VMEM((ECH, DP), jnp.float32),
            pltpu.VMEM((ECH, DP), jnp.float32),
            pltpu.SemaphoreType.DMA,
        ],
    )(_sc_gather_body)
    return f(tab, src, dst)


@functools.partial(jax.jit, static_argnames=())
def kernel(embeddings, hit_score, hit_particle_id, signal_edges, random_edges):
    emb = embeddings.astype(jnp.float32)
    pid = hit_particle_id.astype(jnp.int32)
    pid_row = pid.reshape(N, 1)
    pid_col = pid.reshape(1, N)

    knn_part = pl.pallas_call(
        _knn_body,
        grid=(NSTRIPES,),
        in_specs=[
            pl.BlockSpec((BR, D), lambda i: (i, 0)),
            pl.BlockSpec((N, D), lambda i: (0, 0)),
            pl.BlockSpec((BR, 1), lambda i: (i, 0)),
            pl.BlockSpec((1, N), lambda i: (0, 0)),
        ],
        out_specs=pl.BlockSpec((1, 1, 128), lambda i: (i, 0, 0)),
        out_shape=jax.ShapeDtypeStruct((NSTRIPES, 1, 128), jnp.float32),
        scratch_shapes=[
            pltpu.VMEM((BR, N), jnp.float32),
            pltpu.VMEM((BR, NC2), jnp.float32),
        ],
    )(emb, emb, pid_row, pid_col)

    src = jnp.concatenate([signal_edges[0], random_edges[0]]).astype(jnp.int32)
    dst = jnp.concatenate([signal_edges[1], random_edges[1]]).astype(jnp.int32)
    tab = jnp.concatenate(
        [emb, jax.lax.bitcast_convert_type(pid, jnp.float32).reshape(N, 1),
         jnp.zeros((N, DP - D - 1), jnp.float32)], axis=1)
    es, ed = _sc_gather(tab, src, dst)
    wsig = (jnp.arange(E_ALL) < E_SIG).astype(jnp.float32).reshape(E_ALL, 1)

    edge_part = pl.pallas_call(
        _edge_body,
        grid=(NEBLK,),
        in_specs=[
            pl.BlockSpec((BE, DP), lambda i: (i, 0)),
            pl.BlockSpec((BE, DP), lambda i: (i, 0)),
            pl.BlockSpec((BE, 1), lambda i: (i, 0)),
            pl.BlockSpec((N, 1), lambda i: (0, 0)),
            pl.BlockSpec((N, 1), lambda i: (0, 0)),
        ],
        out_specs=pl.BlockSpec((1, 1, 128), lambda i: (i, 0, 0)),
        out_shape=jax.ShapeDtypeStruct((NEBLK, 1, 128), jnp.float32),
    )(es, ed, wsig, hit_score.astype(jnp.float32).reshape(N, 1), pid_row)

    knn_num = jnp.sum(knn_part[:, 0, 0])
    knn_den = jnp.sum(knn_part[:, 0, 1])
    knn_loss = knn_num / jnp.maximum(knn_den, 1.0)
    sig_sum = jnp.sum(edge_part[:, 0, 0])
    rnd_sum = jnp.sum(edge_part[:, 0, 1])
    beta_sum = jnp.sum(edge_part[:, 0, 2])
    signal_loss = sig_sum / float(E_SIG)
    random_loss = rnd_sum / float(E_RND)
    beta_loss = beta_sum / float(N)
    total = signal_loss + knn_loss + random_loss + beta_loss
    return jnp.stack([total, signal_loss, knn_loss, random_loss, beta_loss])


# speedup vs baseline: 1.0848x; 1.0177x over previous
import functools

import jax
import jax.numpy as jnp
from jax import lax
from jax.experimental import pallas as pl
from jax.experimental.pallas import tpu as pltpu
from jax.experimental.pallas import tpu_sc as plsc

N = 8192
D = 64
E_SIG = 32768
E_RND = 65536
E_ALL = E_SIG + E_RND
K1 = 17
MARGIN = 1.0
R_MAX = 100.0

BR = 256
NSTRIPES = N // BR
NC2 = 2048
BE = 4096
NEBLK = E_ALL // BE


def _knn_body(emb_row_ref, emb_all_ref, pid_row_ref, pid_col_ref, out_ref,
              stripe_ref, cand_ref):
    i = pl.program_id(0)
    a = emb_row_ref[...]
    b = emb_all_ref[...]
    dots = jax.lax.dot_general(
        a, b, (((1,), (1,)), ((), ())),
        preferred_element_type=jnp.float32,
        precision=jax.lax.Precision.HIGHEST)
    sq_r = jnp.sum(a * a, axis=1, keepdims=True)
    sq_c = jax.lax.dot_general(
        jnp.ones((8, D), jnp.float32), b * b, (((1,), (1,)), ((), ())),
        preferred_element_type=jnp.float32,
        precision=jax.lax.Precision.HIGHEST)[:1]
    colio = jax.lax.broadcasted_iota(jnp.int32, (BR, N), 1)
    rowid = i * BR + jax.lax.broadcasted_iota(jnp.int32, (BR, 1), 0)
    pid_r = pid_row_ref[...]
    pid_c = pid_col_ref[...]
    pmi = jnp.where((pid_r == pid_c) & (pid_r != 0), 1, 0)
    packed = jax.lax.bitcast_convert_type(
        (jax.lax.bitcast_convert_type(
            jnp.maximum(sq_r + sq_c - 2.0 * dots, 0.0), jnp.int32) & -2)
        | pmi, jnp.float32)
    stripe_ref[...] = jnp.where(colio == rowid, jnp.float32(3.0e38), packed)

    def rowblk(rb, _):
        Rs = None
        for g in range(4):
            G = [stripe_ref[pl.ds(rb * 8, 8), pl.ds((g * 16 + t) * 128, 128)]
                 for t in range(16)]
            for k in (2, 4, 8, 16):
                j = k >> 1
                while j >= 1:
                    for idx in range(16):
                        l = idx ^ j
                        if l > idx:
                            a, b = G[idx], G[l]
                            lo = jnp.minimum(a, b)
                            hi = jnp.maximum(a, b)
                            if (idx & k) == 0:
                                G[idx], G[l] = lo, hi
                            else:
                                G[idx], G[l] = hi, lo
                    j >>= 1
            if Rs is None:
                Rs = G
            else:
                M = [jnp.minimum(Rs[t], G[15 - t]) for t in range(16)]
                if g < 3:
                    for j in (8, 4, 2, 1):
                        for idx in range(16):
                            l = idx ^ j
                            if l > idx:
                                a, b = M[idx], M[l]
                                M[idx], M[l] = (jnp.minimum(a, b),
                                                jnp.maximum(a, b))
                Rs = M
        for t in range(16):
            cand_ref[pl.ds(rb * 8, 8), pl.ds(t * 128, 128)] = Rs[t]
        return 0

    jax.lax.fori_loop(0, BR // 8, rowblk, 0)

    colio2 = jax.lax.broadcasted_iota(jnp.int32, (BR, NC2), 1)

    def _acc(vmin, num, den):
        pmv = jax.lax.bitcast_convert_type(vmin, jnp.int32) & 1
        d = jnp.sqrt(vmin + 1e-12)
        mf = jnp.where(d <= R_MAX, 1.0, 0.0)
        l = jnp.where(pmv == 1, d, jnp.maximum(0.0, MARGIN - d))
        return num + l * mf, den + mf

    def body(_, carry):
        num, den = carry
        s = cand_ref[...]
        vmin = jnp.min(s, axis=1, keepdims=True)
        amin = jnp.min(jnp.where(s == vmin, colio2, jnp.int32(NC2)),
                       axis=1, keepdims=True)
        cand_ref[...] = jnp.where(colio2 == amin, jnp.float32(3.0e38), s)
        return _acc(vmin, num, den)

    zero = jnp.zeros((BR, 1), jnp.float32)
    num, den = jax.lax.fori_loop(0, K1 - 2, body, (zero, zero))
    num, den = _acc(jnp.min(cand_ref[...], axis=1, keepdims=True), num, den)
    lane = jax.lax.broadcasted_iota(jnp.int32, (1, 1, 128), 2)
    out_ref[...] = jnp.where(lane == 0, jnp.sum(num),
                             jnp.where(lane == 1, jnp.sum(den), 0.0))


def _edge_body(es_ref, ed_ref, wsig_ref, hs_ref, pid_ref, out_ref):
    i = pl.program_id(0)
    es = es_ref[...]
    ed = ed_ref[...]
    laneio = jax.lax.broadcasted_iota(jnp.int32, (BE, DP), 1)
    diff = jnp.where(laneio < D, es - ed, 0.0)
    d = jnp.sqrt(jnp.sum(diff * diff, axis=1, keepdims=True) + 1e-12)
    ps = jax.lax.bitcast_convert_type(es[:, D:D + 1], jnp.int32)
    pd = jax.lax.bitcast_convert_type(ed[:, D:D + 1], jnp.int32)
    y = (ps == pd) & (ps != 0)
    l = jnp.where(y, d, jnp.maximum(0.0, MARGIN - d))
    w = wsig_ref[...]
    ssum = jnp.sum(l * w)
    rsum = jnp.sum(l * (1.0 - w))

    def beta():
        x = hs_ref[...]
        t = jnp.where(pid_ref[...] != 0, 1.0, 0.0)
        bce = (jnp.maximum(x, 0.0) - x * t
               + jnp.log1p(jnp.exp(-jnp.abs(x))))
        return jnp.sum(bce)

    bsum = jax.lax.cond(i == 0, beta, lambda: jnp.float32(0.0))
    lane = jax.lax.broadcasted_iota(jnp.int32, (1, 1, 128), 2)
    out_ref[...] = jnp.where(lane == 0, ssum,
                             jnp.where(lane == 1, rsum,
                                       jnp.where(lane == 2, bsum, 0.0)))


NW = 32
EPW = E_ALL // NW
ECH = 384
NCH = EPW // ECH
DP = 128


def _sc_gather_body(tab_hbm, src_hbm, dst_hbm, es_hbm, ed_hbm,
                    idxs_v, idxd_v, rows_s, rows_d, sem):
    wid = lax.axis_index("s") * 2 + lax.axis_index("c")

    def chunk(c, _):
        base = wid * EPW + c * ECH
        pltpu.sync_copy(src_hbm.at[pl.ds(base, ECH)], idxs_v)
        pltpu.sync_copy(dst_hbm.at[pl.ds(base, ECH)], idxd_v)
        cp1 = pltpu.async_copy(tab_hbm.at[idxs_v], rows_s, sem)
        cp2 = pltpu.async_copy(tab_hbm.at[idxd_v], rows_d, sem)
        cp1.wait()
        cp2.wait()
        pltpu.sync_copy(rows_s, es_hbm.at[pl.ds(base, ECH)])
        pltpu.sync_copy(rows_d, ed_hbm.at[pl.ds(base, ECH)])
        return 0

    lax.fori_loop(0, NCH, chunk, 0)


def _sc_gather(tab, src, dst):
    mesh = plsc.VectorSubcoreMesh(core_axis_name="c", subcore_axis_name="s")
    f = functools.partial(
        pl.kernel, mesh=mesh,
        out_type=[
            jax.ShapeDtypeStruct((E_ALL, DP), jnp.float32),
            jax.ShapeDtypeStruct((E_ALL, DP), jnp.float32),
        ],
        scratch_types=[
            pltpu.VMEM((ECH,), jnp.int32),
            pltpu.VMEM((ECH,), jnp.int32),
            pltpu.VMEM((ECH, DP), jnp.float32),
            pltpu.VMEM((ECH, DP), jnp.float32),
            pltpu.SemaphoreType.DMA,
        ],
    )(_sc_gather_body)
    return f(tab, src, dst)


@functools.partial(jax.jit, static_argnames=())
def kernel(embeddings, hit_score, hit_particle_id, signal_edges, random_edges):
    emb = embeddings.astype(jnp.float32)
    pid = hit_particle_id.astype(jnp.int32)
    pid_row = pid.reshape(N, 1)
    pid_col = pid.reshape(1, N)

    knn_part = pl.pallas_call(
        _knn_body,
        grid=(NSTRIPES,),
        in_specs=[
            pl.BlockSpec((BR, D), lambda i: (i, 0)),
            pl.BlockSpec((N, D), lambda i: (0, 0)),
            pl.BlockSpec((BR, 1), lambda i: (i, 0)),
            pl.BlockSpec((1, N), lambda i: (0, 0)),
        ],
        out_specs=pl.BlockSpec((1, 1, 128), lambda i: (i, 0, 0)),
        out_shape=jax.ShapeDtypeStruct((NSTRIPES, 1, 128), jnp.float32),
        scratch_shapes=[
            pltpu.VMEM((BR, N), jnp.float32),
            pltpu.VMEM((BR, NC2), jnp.float32),
        ],
    )(emb, emb, pid_row, pid_col)

    src = jnp.concatenate([signal_edges[0], random_edges[0]]).astype(jnp.int32)
    dst = jnp.concatenate([signal_edges[1], random_edges[1]]).astype(jnp.int32)
    tab = jnp.concatenate(
        [emb, jax.lax.bitcast_convert_type(pid, jnp.float32).reshape(N, 1),
         jnp.zeros((N, DP - D - 1), jnp.float32)], axis=1)
    es, ed = _sc_gather(tab, src, dst)
    wsig = (jnp.arange(E_ALL) < E_SIG).astype(jnp.float32).reshape(E_ALL, 1)

    edge_part = pl.pallas_call(
        _edge_body,
        grid=(NEBLK,),
        in_specs=[
            pl.BlockSpec((BE, DP), lambda i: (i, 0)),
            pl.BlockSpec((BE, DP), lambda i: (i, 0)),
            pl.BlockSpec((BE, 1), lambda i: (i, 0)),
            pl.BlockSpec((N, 1), lambda i: (0, 0)),
            pl.BlockSpec((N, 1), lambda i: (0, 0)),
        ],
        out_specs=pl.BlockSpec((1, 1, 128), lambda i: (i, 0, 0)),
        out_shape=jax.ShapeDtypeStruct((NEBLK, 1, 128), jnp.float32),
    )(es, ed, wsig, hit_score.astype(jnp.float32).reshape(N, 1), pid_row)

    knn_num = jnp.sum(knn_part[:, 0, 0])
    knn_den = jnp.sum(knn_part[:, 0, 1])
    knn_loss = knn_num / jnp.maximum(knn_den, 1.0)
    sig_sum = jnp.sum(edge_part[:, 0, 0])
    rnd_sum = jnp.sum(edge_part[:, 0, 1])
    beta_sum = jnp.sum(edge_part[:, 0, 2])
    signal_loss = sig_sum / float(E_SIG)
    random_loss = rnd_sum / float(E_RND)
    beta_loss = beta_sum / float(N)
    total = signal_loss + knn_loss + random_loss + beta_loss
    return jnp.stack([total, signal_loss, knn_loss, random_loss, beta_loss])
